# baseline (speedup 1.0000x reference)
import functools

import jax
import jax.numpy as jnp
from jax import lax
from jax.experimental import pallas as pl
from jax.experimental.pallas import tpu as pltpu
from jax.experimental.pallas import tpu_sc as plsc

_NC = 2
_NS = 16
_NW = _NC * _NS
_BLK = 128


def _make_table_body(e, ep):
    pad = ep - e

    def _table_body(z_ref, c_ref, psi_ref, ei_ref, a_ref, row_ref, col_ref):
        w = lax.dot_general(
            z_ref[...], psi_ref[...], (((1,), (1,)), ((), ())),
            preferred_element_type=jnp.float32,
            precision=lax.Precision.HIGHEST,
        )
        a_ref[...] = jnp.concatenate([w, c_ref[...]], axis=1)
        row_ref[pl.ds(0, e)] = ei_ref[0, :]
        col_ref[pl.ds(0, e)] = ei_ref[1, :]
        if pad:
            row_ref[pl.ds(e, pad)] = jnp.zeros((pad,), jnp.int32)
            col_ref[pl.ds(e, pad)] = jnp.zeros((pad,), jnp.int32)

    return _table_body


_DEPTH = 4


def _make_edge_kernel(n, k, bpw):
    ep = _NW * bpw * _BLK
    epw = bpw * _BLK
    chunk = -(-n // _NS)
    mesh = plsc.VectorSubcoreMesh(
        core_axis_name="c", subcore_axis_name="s",
        num_cores=_NC, num_subcores=_NS,
    )

    @functools.partial(
        pl.kernel,
        out_type=jax.ShapeDtypeStruct((ep,), jnp.float32),
        mesh=mesh,
        compiler_params=pltpu.CompilerParams(
            needs_layout_passes=False, use_tc_tiling_on_sc=False),
        scratch_types=[
            pltpu.VMEM_SHARED((n, 2 * k), jnp.float32),
            pltpu.VMEM((epw,), jnp.int32),
            pltpu.VMEM((epw,), jnp.int32),
            [pltpu.VMEM((_BLK, 2 * k), jnp.float32)
             for _ in range(_DEPTH)],
            [pltpu.VMEM((_BLK, 2 * k), jnp.float32)
             for _ in range(_DEPTH)],
            pltpu.VMEM((epw,), jnp.float32),
            [pltpu.SemaphoreType.DMA for _ in range(_DEPTH)],
        ],
    )
    def edge_kernel(a_hbm, row_hbm, col_hbm, out_hbm,
                    tab_sh, ridx_v, cidx_v, ars, acs, out_v, sems):
        sid = lax.axis_index("s")
        wid = sid * _NC + lax.axis_index("c")
        ebase = wid * epw

        rows0 = sid * chunk
        pltpu.sync_copy(a_hbm.at[pl.ds(rows0, chunk)],
                        tab_sh.at[pl.ds(rows0, chunk)])
        pltpu.sync_copy(row_hbm.at[pl.ds(ebase, epw)], ridx_v)
        pltpu.sync_copy(col_hbm.at[pl.ds(ebase, epw)], cidx_v)
        plsc.subcore_barrier()

        def fetch(b, slot):
            pltpu.async_copy(tab_sh.at[ridx_v.at[pl.ds(b * _BLK, _BLK)]],
                             ars[slot], sems[slot])
            pltpu.async_copy(tab_sh.at[cidx_v.at[pl.ds(b * _BLK, _BLK)]],
                             acs[slot], sems[slot])

        def wait(b, slot):
            pltpu.make_async_copy(tab_sh.at[ridx_v.at[pl.ds(b * _BLK, _BLK)]],
                                  ars[slot], sems[slot]).wait()
            pltpu.make_async_copy(tab_sh.at[cidx_v.at[pl.ds(b * _BLK, _BLK)]],
                                  acs[slot], sems[slot]).wait()

        def compute(b, slot):
            ar = ars[slot]
            ac = acs[slot]

            def g_body(g, _):
                lane = lax.broadcasted_iota(jnp.int32, (16,), 0)
                rows = g * 16 + lane
                acc1 = jnp.zeros((16,), jnp.float32)
                acc2 = jnp.zeros((16,), jnp.float32)
                for t in range(k):
                    kv = (lane + t) & (k - 1)
                    kv2 = kv | k
                    w_r = plsc.load_gather(ar, (rows, kv))
                    c_c = plsc.load_gather(ac, (rows, kv2))
                    c_r = plsc.load_gather(ar, (rows, kv2))
                    w_c = plsc.load_gather(ac, (rows, kv))
                    acc1 = acc1 + w_r * c_c
                    acc2 = acc2 + c_r * w_c
                s1 = 1.0 / (1.0 + jnp.exp(-acc1))
                s2 = 1.0 / (1.0 + jnp.exp(-acc2))
                out_v[pl.ds(b * _BLK + g * 16, 16)] = 0.5 * (s1 + s2)
                return _

            lax.fori_loop(0, _BLK // 16, g_body, None, unroll=4)

        for b in range(_DEPTH - 1):
            fetch(b, b)

        def ring_body(j, _):
            for i in range(_DEPTH):
                b = j * _DEPTH + i
                wait(b, i)

                @pl.when(b + _DEPTH - 1 < bpw)
                def _prefetch():
                    fetch(b + _DEPTH - 1, (i + _DEPTH - 1) % _DEPTH)

                compute(b, i)
            return _

        lax.fori_loop(0, bpw // _DEPTH, ring_body, None)
        pltpu.sync_copy(out_v, out_hbm.at[pl.ds(ebase, epw)])

    return edge_kernel


def kernel(z, edge_index, c, psi):
    n, d = z.shape
    k = psi.shape[0]
    e = edge_index.shape[1]

    bpw = -(-e // (_NW * _BLK))
    bpw = -(-bpw // _DEPTH) * _DEPTH
    ep = _NW * bpw * _BLK

    a, row_p, col_p = pl.pallas_call(
        _make_table_body(e, ep),
        out_shape=(
            jax.ShapeDtypeStruct((n, 2 * k), jnp.float32),
            jax.ShapeDtypeStruct((ep,), jnp.int32),
            jax.ShapeDtypeStruct((ep,), jnp.int32),
        ),
    )(z, c, psi, edge_index)

    if n % _NS:
        a = jnp.concatenate(
            [a, jnp.zeros((_NS - n % _NS, 2 * k), jnp.float32)])
    n_p = a.shape[0]

    out = _make_edge_kernel(n_p, k, bpw)(a, row_p, col_p)
    return out[:e]


# speedup vs baseline: 1.0536x; 1.0536x over previous
import functools

import jax
import jax.numpy as jnp
from jax import lax
from jax.experimental import pallas as pl
from jax.experimental.pallas import tpu as pltpu
from jax.experimental.pallas import tpu_sc as plsc

_NC = 2
_NS = 16
_NW = _NC * _NS
_BLK = 128


def _make_table_body(e, ep):
    pad = ep - e

    def _table_body(z_ref, c_ref, psi_ref, ei_ref, a_ref, row_ref, col_ref):
        w = lax.dot_general(
            z_ref[...], psi_ref[...], (((1,), (1,)), ((), ())),
            preferred_element_type=jnp.float32,
            precision=lax.Precision.HIGHEST,
        )
        a_ref[...] = jnp.concatenate([w, c_ref[...]], axis=1)
        row_ref[pl.ds(0, e)] = ei_ref[0, :]
        col_ref[pl.ds(0, e)] = ei_ref[1, :]
        if pad:
            row_ref[pl.ds(e, pad)] = jnp.zeros((pad,), jnp.int32)
            col_ref[pl.ds(e, pad)] = jnp.zeros((pad,), jnp.int32)

    return _table_body


_DEPTH = 4


def _make_edge_kernel(n, k, bpw):
    ep = _NW * bpw * _BLK
    epw = bpw * _BLK
    chunk = -(-n // _NS)
    mesh = plsc.VectorSubcoreMesh(
        core_axis_name="c", subcore_axis_name="s",
        num_cores=_NC, num_subcores=_NS,
    )

    @functools.partial(
        pl.kernel,
        out_type=jax.ShapeDtypeStruct((ep,), jnp.float32),
        mesh=mesh,
        compiler_params=pltpu.CompilerParams(
            needs_layout_passes=False, use_tc_tiling_on_sc=False),
        scratch_types=[
            pltpu.VMEM_SHARED((n, 2 * k), jnp.float32),
            pltpu.VMEM((epw,), jnp.int32),
            pltpu.VMEM((epw,), jnp.int32),
            [pltpu.VMEM((_BLK, 2 * k), jnp.float32)
             for _ in range(_DEPTH)],
            [pltpu.VMEM((_BLK, 2 * k), jnp.float32)
             for _ in range(_DEPTH)],
            pltpu.VMEM((epw,), jnp.float32),
            pltpu.VMEM((_BLK,), jnp.float32),
            pltpu.VMEM((_BLK,), jnp.float32),
            [pltpu.SemaphoreType.DMA for _ in range(_DEPTH)],
            pltpu.SemaphoreType.DMA,
        ],
    )
    def edge_kernel(a_hbm, row_hbm, col_hbm, out_hbm,
                    tab_sh, ridx_v, cidx_v, ars, acs, out_v, l1_v, l2_v,
                    sems, sem_st):
        sid = lax.axis_index("s")
        wid = sid * _NC + lax.axis_index("c")
        ebase = wid * epw

        rows0 = sid * chunk
        st1 = pltpu.async_copy(a_hbm.at[pl.ds(rows0, chunk)],
                               tab_sh.at[pl.ds(rows0, chunk)], sem_st)
        st2 = pltpu.async_copy(row_hbm.at[pl.ds(ebase, epw)], ridx_v, sem_st)
        st3 = pltpu.async_copy(col_hbm.at[pl.ds(ebase, epw)], cidx_v, sem_st)
        st1.wait()
        st2.wait()
        st3.wait()
        plsc.subcore_barrier()

        def fetch(b, slot):
            pltpu.async_copy(tab_sh.at[ridx_v.at[pl.ds(b * _BLK, _BLK)]],
                             ars[slot], sems[slot])
            pltpu.async_copy(tab_sh.at[cidx_v.at[pl.ds(b * _BLK, _BLK)]],
                             acs[slot], sems[slot])

        def wait(b, slot):
            pltpu.make_async_copy(tab_sh.at[ridx_v.at[pl.ds(b * _BLK, _BLK)]],
                                  ars[slot], sems[slot]).wait()
            pltpu.make_async_copy(tab_sh.at[cidx_v.at[pl.ds(b * _BLK, _BLK)]],
                                  acs[slot], sems[slot]).wait()

        def compute(b, slot):
            ar = ars[slot]
            ac = acs[slot]

            def g_body(g, _):
                lane = lax.broadcasted_iota(jnp.int32, (16,), 0)
                rows = g * 16 + lane
                acc1 = jnp.zeros((16,), jnp.float32)
                acc2 = jnp.zeros((16,), jnp.float32)
                for t in range(k):
                    kv = (lane + t) & (k - 1)
                    kv2 = kv | k
                    w_r = plsc.load_gather(ar, (rows, kv))
                    c_c = plsc.load_gather(ac, (rows, kv2))
                    c_r = plsc.load_gather(ar, (rows, kv2))
                    w_c = plsc.load_gather(ac, (rows, kv))
                    acc1 = acc1 + w_r * c_c
                    acc2 = acc2 + c_r * w_c
                l1_v[pl.ds(g * 16, 16)] = acc1
                l2_v[pl.ds(g * 16, 16)] = acc2
                return _

            lax.fori_loop(0, _BLK // 16, g_body, None, unroll=2)

            def s_body(g, _):
                l1 = l1_v[pl.ds(g * 16, 16)]
                l2 = l2_v[pl.ds(g * 16, 16)]
                s1 = 1.0 / (1.0 + jnp.exp(-l1))
                s2 = 1.0 / (1.0 + jnp.exp(-l2))
                out_v[pl.ds(b * _BLK + g * 16, 16)] = 0.5 * (s1 + s2)
                return _

            lax.fori_loop(0, _BLK // 16, s_body, None, unroll=2)

        for b in range(_DEPTH - 1):
            fetch(b, b)

        def ring_body(j, _):
            for i in range(_DEPTH):
                b = j * _DEPTH + i
                wait(b, i)

                @pl.when(b + _DEPTH - 1 < bpw)
                def _prefetch():
                    fetch(b + _DEPTH - 1, (i + _DEPTH - 1) % _DEPTH)

                compute(b, i)
            return _

        lax.fori_loop(0, bpw // _DEPTH, ring_body, None)
        pltpu.sync_copy(out_v, out_hbm.at[pl.ds(ebase, epw)])

    return edge_kernel


def kernel(z, edge_index, c, psi):
    n, d = z.shape
    k = psi.shape[0]
    e = edge_index.shape[1]

    bpw = -(-e // (_NW * _BLK))
    bpw = -(-bpw // _DEPTH) * _DEPTH
    ep = _NW * bpw * _BLK

    a, row_p, col_p = pl.pallas_call(
        _make_table_body(e, ep),
        out_shape=(
            jax.ShapeDtypeStruct((n, 2 * k), jnp.float32),
            jax.ShapeDtypeStruct((ep,), jnp.int32),
            jax.ShapeDtypeStruct((ep,), jnp.int32),
        ),
    )(z, c, psi, edge_index)

    if n % _NS:
        a = jnp.concatenate(
            [a, jnp.zeros((_NS - n % _NS, 2 * k), jnp.float32)])
    n_p = a.shape[0]

    out = _make_edge_kernel(n_p, k, bpw)(a, row_p, col_p)
    return out[:e]


# speedup vs baseline: 1.1424x; 1.0843x over previous
import functools

import jax
import jax.numpy as jnp
from jax import lax
from jax.experimental import pallas as pl
from jax.experimental.pallas import tpu as pltpu
from jax.experimental.pallas import tpu_sc as plsc

_NC = 2
_NS = 16
_NW = _NC * _NS
_BLK = 128


def _make_table_body(e, ep):
    pad = ep - e

    def _table_body(z_ref, c_ref, psi_ref, ei_ref, a_ref, row_ref, col_ref):
        w = lax.dot_general(
            z_ref[...], psi_ref[...], (((1,), (1,)), ((), ())),
            preferred_element_type=jnp.float32,
            precision=lax.Precision.HIGHEST,
        )
        a_ref[...] = jnp.concatenate([w, c_ref[...]], axis=1)
        row_ref[pl.ds(0, e)] = ei_ref[0, :]
        col_ref[pl.ds(0, e)] = ei_ref[1, :]
        if pad:
            row_ref[pl.ds(e, pad)] = jnp.zeros((pad,), jnp.int32)
            col_ref[pl.ds(e, pad)] = jnp.zeros((pad,), jnp.int32)

    return _table_body


_DEPTH = 4


def _make_edge_kernel(n, k, bpw):
    ep = _NW * bpw * _BLK
    epw = bpw * _BLK
    chunk = -(-n // _NS)
    mesh = plsc.VectorSubcoreMesh(
        core_axis_name="c", subcore_axis_name="s",
        num_cores=_NC, num_subcores=_NS,
    )

    @functools.partial(
        pl.kernel,
        out_type=jax.ShapeDtypeStruct((ep,), jnp.float32),
        mesh=mesh,
        compiler_params=pltpu.CompilerParams(
            needs_layout_passes=False, use_tc_tiling_on_sc=False),
        scratch_types=[
            pltpu.VMEM_SHARED((n, 2 * k), jnp.float32),
            pltpu.VMEM((epw,), jnp.int32),
            pltpu.VMEM((epw,), jnp.int32),
            [pltpu.VMEM((_BLK, 2 * k), jnp.float32)
             for _ in range(_DEPTH)],
            [pltpu.VMEM((_BLK, 2 * k), jnp.float32)
             for _ in range(_DEPTH)],
            pltpu.VMEM((epw,), jnp.float32),
            [pltpu.SemaphoreType.DMA for _ in range(_DEPTH)],
            pltpu.SemaphoreType.DMA,
        ],
    )
    def edge_kernel(a_hbm, row_hbm, col_hbm, out_hbm,
                    tab_sh, ridx_v, cidx_v, ars, acs, out_v,
                    sems, sem_st):
        sid = lax.axis_index("s")
        wid = sid * _NC + lax.axis_index("c")
        ebase = wid * epw

        rows0 = sid * chunk
        st1 = pltpu.async_copy(a_hbm.at[pl.ds(rows0, chunk)],
                               tab_sh.at[pl.ds(rows0, chunk)], sem_st)
        st2 = pltpu.async_copy(row_hbm.at[pl.ds(ebase, epw)], ridx_v, sem_st)
        st3 = pltpu.async_copy(col_hbm.at[pl.ds(ebase, epw)], cidx_v, sem_st)
        st1.wait()
        st2.wait()
        st3.wait()
        plsc.subcore_barrier()

        def fetch(b, slot):
            pltpu.async_copy(tab_sh.at[ridx_v.at[pl.ds(b * _BLK, _BLK)]],
                             ars[slot], sems[slot])
            pltpu.async_copy(tab_sh.at[cidx_v.at[pl.ds(b * _BLK, _BLK)]],
                             acs[slot], sems[slot])

        def wait(b, slot):
            pltpu.make_async_copy(tab_sh.at[ridx_v.at[pl.ds(b * _BLK, _BLK)]],
                                  ars[slot], sems[slot]).wait()
            pltpu.make_async_copy(tab_sh.at[cidx_v.at[pl.ds(b * _BLK, _BLK)]],
                                  acs[slot], sems[slot]).wait()

        def compute(b, slot):
            ar = ars[slot]
            ac = acs[slot]

            def g_body(g, _):
                lane = lax.broadcasted_iota(jnp.int32, (16,), 0)
                rows = g * 16 + lane
                acc1 = jnp.zeros((16,), jnp.float32)
                acc2 = jnp.zeros((16,), jnp.float32)
                for t in range(k):
                    kv = (lane + t) & (k - 1)
                    kv2 = kv | k
                    w_r = plsc.load_gather(ar, (rows, kv))
                    c_c = plsc.load_gather(ac, (rows, kv2))
                    c_r = plsc.load_gather(ar, (rows, kv2))
                    w_c = plsc.load_gather(ac, (rows, kv))
                    acc1 = acc1 + w_r * c_c
                    acc2 = acc2 + c_r * w_c
                s1 = 1.0 / (1.0 + jnp.exp(-acc1))
                s2 = 1.0 / (1.0 + jnp.exp(-acc2))
                out_v[pl.ds(b * _BLK + g * 16, 16)] = 0.5 * (s1 + s2)
                return _

            lax.fori_loop(0, _BLK // 16, g_body, None, unroll=2)

        for b in range(_DEPTH - 1):
            fetch(b, b)

        def ring_body(j, _):
            for i in range(_DEPTH):
                b = j * _DEPTH + i
                wait(b, i)

                @pl.when(b + _DEPTH - 1 < bpw)
                def _prefetch():
                    fetch(b + _DEPTH - 1, (i + _DEPTH - 1) % _DEPTH)

                compute(b, i)
            return _

        lax.fori_loop(0, bpw // _DEPTH, ring_body, None)
        pltpu.sync_copy(out_v, out_hbm.at[pl.ds(ebase, epw)])

    return edge_kernel


def kernel(z, edge_index, c, psi):
    n, d = z.shape
    k = psi.shape[0]
    e = edge_index.shape[1]

    bpw = -(-e // (_NW * _BLK))
    bpw = -(-bpw // _DEPTH) * _DEPTH
    ep = _NW * bpw * _BLK

    a, row_p, col_p = pl.pallas_call(
        _make_table_body(e, ep),
        out_shape=(
            jax.ShapeDtypeStruct((n, 2 * k), jnp.float32),
            jax.ShapeDtypeStruct((ep,), jnp.int32),
            jax.ShapeDtypeStruct((ep,), jnp.int32),
        ),
    )(z, c, psi, edge_index)

    if n % _NS:
        a = jnp.concatenate(
            [a, jnp.zeros((_NS - n % _NS, 2 * k), jnp.float32)])
    n_p = a.shape[0]

    out = _make_edge_kernel(n_p, k, bpw)(a, row_p, col_p)
    return out[:e]


# speedup vs baseline: 1.1456x; 1.0028x over previous
import functools

import jax
import jax.numpy as jnp
from jax import lax
from jax.experimental import pallas as pl
from jax.experimental.pallas import tpu as pltpu
from jax.experimental.pallas import tpu_sc as plsc

_NC = 2
_NS = 16
_NW = _NC * _NS
_BLK = 128


def _make_table_body(e, ep):
    pad = ep - e

    def _table_body(z_ref, c_ref, psi_ref, ei_ref, a_ref, row_ref, col_ref):
        w = lax.dot_general(
            z_ref[...], psi_ref[...], (((1,), (1,)), ((), ())),
            preferred_element_type=jnp.float32,
            precision=lax.Precision.HIGHEST,
        )
        a_ref[...] = jnp.concatenate([w, c_ref[...]], axis=1)
        row_ref[pl.ds(0, e)] = ei_ref[0, :]
        col_ref[pl.ds(0, e)] = ei_ref[1, :]
        if pad:
            row_ref[pl.ds(e, pad)] = jnp.zeros((pad,), jnp.int32)
            col_ref[pl.ds(e, pad)] = jnp.zeros((pad,), jnp.int32)

    return _table_body


_DEPTH = 4


def _make_edge_kernel(n, k, bpw):
    ep = _NW * bpw * _BLK
    epw = bpw * _BLK
    chunk = -(-n // _NS)
    mesh = plsc.VectorSubcoreMesh(
        core_axis_name="c", subcore_axis_name="s",
        num_cores=_NC, num_subcores=_NS,
    )

    @functools.partial(
        pl.kernel,
        out_type=jax.ShapeDtypeStruct((ep,), jnp.float32),
        mesh=mesh,
        compiler_params=pltpu.CompilerParams(
            needs_layout_passes=False, use_tc_tiling_on_sc=False),
        scratch_types=[
            pltpu.VMEM_SHARED((n, 2 * k), jnp.float32),
            pltpu.VMEM((epw,), jnp.int32),
            pltpu.VMEM((epw,), jnp.int32),
            [pltpu.VMEM((_BLK, 2 * k), jnp.float32)
             for _ in range(_DEPTH)],
            [pltpu.VMEM((_BLK, 2 * k), jnp.float32)
             for _ in range(_DEPTH)],
            pltpu.VMEM((epw,), jnp.float32),
            [pltpu.SemaphoreType.DMA for _ in range(_DEPTH)],
            pltpu.SemaphoreType.DMA,
        ],
    )
    def edge_kernel(a_hbm, row_hbm, col_hbm, out_hbm,
                    tab_sh, ridx_v, cidx_v, ars, acs, out_v,
                    sems, sem_st):
        sid = lax.axis_index("s")
        wid = sid * _NC + lax.axis_index("c")
        ebase = wid * epw

        rows0 = sid * chunk
        st1 = pltpu.async_copy(a_hbm.at[pl.ds(rows0, chunk)],
                               tab_sh.at[pl.ds(rows0, chunk)], sem_st)
        st2 = pltpu.async_copy(row_hbm.at[pl.ds(ebase, epw)], ridx_v, sem_st)
        st3 = pltpu.async_copy(col_hbm.at[pl.ds(ebase, epw)], cidx_v, sem_st)
        st1.wait()
        st2.wait()
        st3.wait()
        plsc.subcore_barrier()

        def fetch(b, slot):
            pltpu.async_copy(tab_sh.at[ridx_v.at[pl.ds(b * _BLK, _BLK)]],
                             ars[slot], sems[slot])
            pltpu.async_copy(tab_sh.at[cidx_v.at[pl.ds(b * _BLK, _BLK)]],
                             acs[slot], sems[slot])

        def wait(b, slot):
            pltpu.make_async_copy(tab_sh.at[ridx_v.at[pl.ds(b * _BLK, _BLK)]],
                                  ars[slot], sems[slot]).wait()
            pltpu.make_async_copy(tab_sh.at[cidx_v.at[pl.ds(b * _BLK, _BLK)]],
                                  acs[slot], sems[slot]).wait()

        def compute(b, slot):
            ar = ars[slot]
            ac = acs[slot]

            def g_body(g, _):
                lane = lax.broadcasted_iota(jnp.int32, (16,), 0)
                rows = g * 16 + lane
                acc1 = jnp.zeros((16,), jnp.float32)
                acc2 = jnp.zeros((16,), jnp.float32)
                for t in range(k):
                    kv = (lane + t) & (k - 1)
                    kv2 = kv | k
                    w_r = plsc.load_gather(ar, (rows, kv))
                    c_c = plsc.load_gather(ac, (rows, kv2))
                    c_r = plsc.load_gather(ar, (rows, kv2))
                    w_c = plsc.load_gather(ac, (rows, kv))
                    acc1 = acc1 + w_r * c_c
                    acc2 = acc2 + c_r * w_c
                e1 = jnp.exp(-jnp.maximum(acc1, -30.0))
                e2 = jnp.exp(-jnp.maximum(acc2, -30.0))
                num = 1.0 + 0.5 * (e1 + e2)
                den = (1.0 + e1) * (1.0 + e2)
                out_v[pl.ds(b * _BLK + g * 16, 16)] = num / den
                return _

            lax.fori_loop(0, _BLK // 16, g_body, None, unroll=2)

        for b in range(_DEPTH - 1):
            fetch(b, b)

        def ring_body(j, _):
            for i in range(_DEPTH):
                b = j * _DEPTH + i
                wait(b, i)

                @pl.when(b + _DEPTH - 1 < bpw)
                def _prefetch():
                    fetch(b + _DEPTH - 1, (i + _DEPTH - 1) % _DEPTH)

                compute(b, i)
            return _

        lax.fori_loop(0, bpw // _DEPTH, ring_body, None)
        pltpu.sync_copy(out_v, out_hbm.at[pl.ds(ebase, epw)])

    return edge_kernel


def kernel(z, edge_index, c, psi):
    n, d = z.shape
    k = psi.shape[0]
    e = edge_index.shape[1]

    bpw = -(-e // (_NW * _BLK))
    bpw = -(-bpw // _DEPTH) * _DEPTH
    ep = _NW * bpw * _BLK

    a, row_p, col_p = pl.pallas_call(
        _make_table_body(e, ep),
        out_shape=(
            jax.ShapeDtypeStruct((n, 2 * k), jnp.float32),
            jax.ShapeDtypeStruct((ep,), jnp.int32),
            jax.ShapeDtypeStruct((ep,), jnp.int32),
        ),
    )(z, c, psi, edge_index)

    if n % _NS:
        a = jnp.concatenate(
            [a, jnp.zeros((_NS - n % _NS, 2 * k), jnp.float32)])
    n_p = a.shape[0]

    out = _make_edge_kernel(n_p, k, bpw)(a, row_p, col_p)
    return out[:e]
